# baseline (speedup 1.0000x reference)
import functools
import math

import jax
import jax.numpy as jnp
from jax import lax
from jax.experimental import pallas as pl
from jax.experimental.pallas import tpu as pltpu
from jax.experimental.pallas import tpu_sc as plsc

N = 512
SEQ = 12
FIN = 64
HID = 128
OUT = 64
E = 16384

NC = 2
NS = 16
NW = NC * NS
EPW = E // NW
ROWS = EPW // 128
CELLS = N * N
CPS = CELLS // NS
ZCH = 2048


def _edge_scatter_body(src_hbm, dst_hbm, w_hbm, wp_hbm, cp_hbm,
                       src_v, dst_v, w_v, idx_v, ones_v, stage_v, wsh, csh):
    c = lax.axis_index("c")
    s = lax.axis_index("s")
    wid = s * NC + c
    pltpu.sync_copy(src_hbm.at[pl.ds(wid * ROWS, ROWS)], src_v)
    pltpu.sync_copy(dst_hbm.at[pl.ds(wid * ROWS, ROWS)], dst_v)
    pltpu.sync_copy(w_hbm.at[pl.ds(wid * ROWS, ROWS)], w_v)

    def zbody(i, carry):
        stage_v[pl.ds(i * 16, 16)] = jnp.zeros((16,), jnp.float32)
        return carry
    lax.fori_loop(0, ZCH // 16, zbody, 0)
    for k in range(CPS // ZCH):
        pltpu.sync_copy(stage_v, wsh.at[pl.ds(s * CPS + k * ZCH, ZCH)])
        pltpu.sync_copy(stage_v, csh.at[pl.ds(s * CPS + k * ZCH, ZCH)])

    for r in range(ROWS):
        for ch in range(8):
            sl = pl.ds(ch * 16, 16)
            idx_v[r, sl] = dst_v[r, sl] * N + src_v[r, sl]
            ones_v[r, sl] = jnp.full((16,), 1.0, jnp.float32)

    plsc.subcore_barrier()
    for r in range(ROWS):
        pltpu.sync_copy(w_v.at[r], wsh.at[idx_v.at[r]], add=True)
        pltpu.sync_copy(ones_v.at[r], csh.at[idx_v.at[r]], add=True)
    plsc.subcore_barrier()

    pltpu.sync_copy(wsh.at[pl.ds(s * CPS, CPS)], wp_hbm.at[c, pl.ds(s * CPS, CPS)])
    pltpu.sync_copy(csh.at[pl.ds(s * CPS, CPS)], cp_hbm.at[c, pl.ds(s * CPS, CPS)])


@functools.cache
def _edge_scatter():
    return pl.kernel(
        _edge_scatter_body,
        mesh=plsc.VectorSubcoreMesh(core_axis_name="c", subcore_axis_name="s"),
        out_type=[jax.ShapeDtypeStruct((NC, CELLS), jnp.float32),
                  jax.ShapeDtypeStruct((NC, CELLS), jnp.float32)],
        scratch_types=[
            pltpu.VMEM((ROWS, 128), jnp.int32),
            pltpu.VMEM((ROWS, 128), jnp.int32),
            pltpu.VMEM((ROWS, 128), jnp.float32),
            pltpu.VMEM((ROWS, 128), jnp.int32),
            pltpu.VMEM((ROWS, 128), jnp.float32),
            pltpu.VMEM((ZCH,), jnp.float32),
            pltpu.VMEM_SHARED((CELLS,), jnp.float32),
            pltpu.VMEM_SHARED((CELLS,), jnp.float32),
        ],
    )


def _layer_norm(h, g, b):
    mu = jnp.mean(h, axis=1, keepdims=True)
    d = h - mu
    var = jnp.mean(d * d, axis=1, keepdims=True)
    return d * lax.rsqrt(var + 1e-5) * g + b


def _tconv(h, C, Wq, bq, Wk, bk, Wv, bv, Ws, bs):
    q = jnp.dot(h, Wq) + bq
    k = jnp.dot(h, Wk) + bk
    v = jnp.dot(h, Wv) + bv
    sc = lax.dot_general(q, k, (((1,), (1,)), ((), ()))) * (1.0 / math.sqrt(HID))
    neg = jnp.where(C > 0, sc, -1e30)
    m = jnp.max(neg, axis=1, keepdims=True)
    m = jnp.where(m > -1e29, m, 0.0)
    ee = C * jnp.exp(jnp.minimum(sc - m, 0.0))
    denom = jnp.sum(ee, axis=1, keepdims=True)
    msg = jnp.dot(ee, v)
    return msg / (denom + 1e-16) + jnp.dot(h, Ws) + bs


def _dense_body(*refs):
    (wp, cp, x2d,
     Wzc, Wzl, bzc, bzl, Whc, Whl, bhc, bhl,
     projW, projb, convk, convb,
     q1W, q1b, k1W, k1b, v1W, v1b, s1W, s1b, ln1g, ln1b,
     q2W, q2b, k2W, k2b, v2W, v2b, s2W, s2b, ln2g, ln2b,
     skW, skb, predW, predb, ip1W, ip1b) = [r[...] for r in refs[:-3]]
    npred_out, ait_out, bjt_out = refs[-3:]
    W = wp[0] + wp[1]
    C = cp[0] + cp[1]
    deg = jnp.sum(W, axis=1, keepdims=True) + 1.0
    dinv = lax.rsqrt(deg)
    y = x2d * dinv
    S = (jnp.dot(W, y) + y) * dinv

    Wz = jnp.dot(Wzc, Wzl[:HID, :])
    bz = jnp.dot(bzc, Wzl[:HID, :]) + bzl
    Wh = jnp.dot(Whc, Whl[:HID, :])
    bh = jnp.dot(bhc, Whl[:HID, :]) + bhl

    hsum = jnp.zeros((N, HID), jnp.float32)
    h0 = None
    hlast = None
    for t in range(SEQ):
        St = S[:, t * FIN:(t + 1) * FIN]
        Zt = jax.nn.sigmoid(jnp.dot(St, Wz) + bz)
        Tt = jnp.tanh(jnp.dot(St, Wh) + bh)
        ht = (1.0 - Zt) * Tt
        if t == 0:
            h0 = ht
        if t == SEQ - 1:
            hlast = ht
        hsum = hsum + ht

    Psum = jnp.dot(hsum, projW) + SEQ * projb
    P0 = jnp.dot(h0, projW) + projb
    PL = jnp.dot(hlast, projW) + projb
    ht_mean = (jnp.dot(Psum - PL, convk[0]) + jnp.dot(Psum, convk[1])
               + jnp.dot(Psum - P0, convk[2])) * (1.0 / SEQ) + convb

    hi = hsum * (1.0 / SEQ)
    hi = _tconv(hi, C, q1W, q1b, k1W, k1b, v1W, v1b, s1W, s1b)
    hi = jnp.maximum(_layer_norm(hi, ln1g, ln1b), 0.0)
    hi = _tconv(hi, C, q2W, q2b, k2W, k2b, v2W, v2b, s2W, s2b)
    hi = jnp.maximum(_layer_norm(hi, ln2g, ln2b), 0.0)
    hi = hi + jnp.dot(hi, skW) + skb

    hc = jnp.concatenate([ht_mean, hi], axis=1)
    npred_out[...] = jnp.dot(hc, predW) + predb
    ait_out[...] = lax.dot_general(
        ip1W[:2 * HID, :], hc, (((0,), (1,)), ((), ()))) + ip1b
    bjt_out[...] = lax.dot_general(
        ip1W[2 * HID:, :], hc, (((0,), (1,)), ((), ())))


TI = 32


def _pair_body(ait, bjt, w2, b2t, w3t, b3, out):
    at = ait[0]
    bt = bjt[...]
    zero = jnp.bfloat16(0.0)
    h1 = jnp.maximum(at + bt, zero)
    z = lax.dot_general(w2[...], h1, (((0,), (0,)), ((), ())),
                        preferred_element_type=jnp.float32)
    h2 = jnp.maximum(z + b2t[...][:, :, None], 0.0)
    r = jnp.sum(h2 * w3t[...][:, :, None], axis=0) + b3[0, 0]
    out[...] = jax.nn.sigmoid(r)


def kernel(x, edge_index, edge_weight, params):
    p = params
    t = p['tgcn']
    tc1 = p['tc1']
    tc2 = p['tc2']

    src = edge_index[0].reshape(E // 128, 128)
    dst = edge_index[1].reshape(E // 128, 128)
    ew = edge_weight.reshape(E // 128, 128)
    wp, cp = _edge_scatter()(src, dst, ew)
    wp = wp.reshape(NC, N, N)
    cp = cp.reshape(NC, N, N)

    x2d = jnp.transpose(x, (1, 0, 2)).reshape(N, SEQ * FIN)
    convk = jnp.transpose(p['conv_W'], (2, 1, 0))
    r2 = lambda v: v.reshape(1, -1)

    npred, ait, bjt = pl.pallas_call(
        _dense_body,
        out_shape=[
            jax.ShapeDtypeStruct((N, OUT), jnp.float32),
            jax.ShapeDtypeStruct((HID, N), jnp.float32),
            jax.ShapeDtypeStruct((HID, N), jnp.float32),
        ],
    )(
        wp, cp, x2d,
        t['Wzc'], t['Wzl'], r2(t['bzc']), r2(t['bzl']),
        t['Whc'], t['Whl'], r2(t['bhc']), r2(t['bhl']),
        p['proj_W'], r2(p['proj_b']), convk, r2(p['conv_b']),
        tc1['Wq'], r2(tc1['bq']), tc1['Wk'], r2(tc1['bk']),
        tc1['Wv'], r2(tc1['bv']), tc1['Ws'], r2(tc1['bs']),
        r2(p['ln1_g']), r2(p['ln1_b']),
        tc2['Wq'], r2(tc2['bq']), tc2['Wk'], r2(tc2['bk']),
        tc2['Wv'], r2(tc2['bv']), tc2['Ws'], r2(tc2['bs']),
        r2(p['ln2_g']), r2(p['ln2_b']),
        p['skip_W'], r2(p['skip_b']), p['pred_W'], r2(p['pred_b']),
        p['ip1_W'], p['ip1_b'].reshape(-1, 1),
    )

    inter = pl.pallas_call(
        _pair_body,
        grid=(N // TI,),
        in_specs=[
            pl.BlockSpec((1, HID, TI, 1), lambda i: (i, 0, 0, 0)),
            pl.BlockSpec((HID, 1, N), lambda i: (0, 0, 0)),
            pl.BlockSpec((HID, HID // 2), lambda i: (0, 0)),
            pl.BlockSpec((HID // 2, 1), lambda i: (0, 0)),
            pl.BlockSpec((HID // 2, 1), lambda i: (0, 0)),
            pl.BlockSpec((1, 1), lambda i: (0, 0)),
        ],
        out_specs=pl.BlockSpec((TI, N), lambda i: (i, 0)),
        out_shape=jax.ShapeDtypeStruct((N, N), jnp.float32),
    )(ait.astype(jnp.bfloat16).reshape(HID, N // TI, TI)
        .transpose(1, 0, 2).reshape(N // TI, HID, TI, 1),
      bjt.astype(jnp.bfloat16).reshape(HID, 1, N),
      p['ip2_W'].astype(jnp.bfloat16), p['ip2_b'].reshape(-1, 1),
      p['ip3_W'], p['ip3_b'].reshape(1, 1))

    return npred, inter


# speedup vs baseline: 1.9491x; 1.9491x over previous
import functools
import math

import jax
import jax.numpy as jnp
from jax import lax
from jax.experimental import pallas as pl
from jax.experimental.pallas import tpu as pltpu
from jax.experimental.pallas import tpu_sc as plsc

N = 512
SEQ = 12
FIN = 64
HID = 128
OUT = 64
E = 16384

NC = 2
NS = 16
ROWS = E // NS // 128
CELLS = N * N
CPS = CELLS // NS
ZCH = 2048


def _edge_scatter_body(src_hbm, dst_hbm, w_hbm, wp_hbm, cp_hbm,
                       src_v, dst_v, w_v, idx_v, stage_v, sh):
    c = lax.axis_index("c")
    s = lax.axis_index("s")
    pltpu.sync_copy(src_hbm.at[pl.ds(s * ROWS, ROWS)], src_v)
    pltpu.sync_copy(dst_hbm.at[pl.ds(s * ROWS, ROWS)], dst_v)

    @pl.when(c == 0)
    def _():
        pltpu.sync_copy(w_hbm.at[pl.ds(s * ROWS, ROWS)], w_v)

    @pl.when(c == 1)
    def _():
        for r in range(ROWS):
            for ch in range(8):
                w_v[r, pl.ds(ch * 16, 16)] = jnp.full((16,), 1.0, jnp.float32)

    for r in range(ROWS):
        for ch in range(8):
            sl = pl.ds(ch * 16, 16)
            idx_v[r, sl] = dst_v[r, sl] * N + src_v[r, sl]

    def zbody(i, carry):
        stage_v[pl.ds(i * 16, 16)] = jnp.zeros((16,), jnp.float32)
        return carry
    lax.fori_loop(0, ZCH // 16, zbody, 0)
    for k in range(CPS // ZCH):
        pltpu.sync_copy(stage_v, sh.at[pl.ds(s * CPS + k * ZCH, ZCH)])

    plsc.subcore_barrier()
    for r in range(ROWS):
        pltpu.sync_copy(w_v.at[r], sh.at[idx_v.at[r]], add=True)
    plsc.subcore_barrier()

    @pl.when(c == 0)
    def _():
        pltpu.sync_copy(sh.at[pl.ds(s * CPS, CPS)],
                        wp_hbm.at[pl.ds(s * CPS, CPS)])

    @pl.when(c == 1)
    def _():
        pltpu.sync_copy(sh.at[pl.ds(s * CPS, CPS)],
                        cp_hbm.at[pl.ds(s * CPS, CPS)])


@functools.cache
def _edge_scatter():
    return pl.kernel(
        _edge_scatter_body,
        mesh=plsc.VectorSubcoreMesh(core_axis_name="c", subcore_axis_name="s"),
        out_type=[jax.ShapeDtypeStruct((CELLS,), jnp.float32),
                  jax.ShapeDtypeStruct((CELLS,), jnp.float32)],
        scratch_types=[
            pltpu.VMEM((ROWS, 128), jnp.int32),
            pltpu.VMEM((ROWS, 128), jnp.int32),
            pltpu.VMEM((ROWS, 128), jnp.float32),
            pltpu.VMEM((ROWS, 128), jnp.int32),
            pltpu.VMEM((ZCH,), jnp.float32),
            pltpu.VMEM_SHARED((CELLS,), jnp.float32),
        ],
    )


def _layer_norm(h, g, b):
    mu = jnp.mean(h, axis=1, keepdims=True)
    d = h - mu
    var = jnp.mean(d * d, axis=1, keepdims=True)
    return d * lax.rsqrt(var + 1e-5) * g + b


def _tconv(h, C, Wq, bq, Wk, bk, Wv, bv, Ws, bs):
    q = jnp.dot(h, Wq) + bq
    k = jnp.dot(h, Wk) + bk
    v = jnp.dot(h, Wv) + bv
    sc = lax.dot_general(q, k, (((1,), (1,)), ((), ()))) * (1.0 / math.sqrt(HID))
    neg = jnp.where(C > 0, sc, -1e30)
    m = jnp.max(neg, axis=1, keepdims=True)
    m = jnp.where(m > -1e29, m, 0.0)
    ee = C * jnp.exp(jnp.minimum(sc - m, 0.0))
    denom = jnp.sum(ee, axis=1, keepdims=True)
    msg = jnp.dot(ee, v)
    return msg / (denom + 1e-16) + jnp.dot(h, Ws) + bs


def _dense_body(*refs):
    (wp, cp, x2d,
     Wzc, Wzl, bzc, bzl, Whc, Whl, bhc, bhl,
     projW, projb, convk, convb,
     q1W, q1b, k1W, k1b, v1W, v1b, s1W, s1b, ln1g, ln1b,
     q2W, q2b, k2W, k2b, v2W, v2b, s2W, s2b, ln2g, ln2b,
     skW, skb, predW, predb, ip1W, ip1b) = [r[...] for r in refs[:-3]]
    npred_out, ait_out, bjt_out = refs[-3:]
    W = wp
    C = cp
    deg = jnp.sum(W, axis=1, keepdims=True) + 1.0
    dinv = lax.rsqrt(deg)
    y = x2d * dinv
    S = (jnp.dot(W, y) + y) * dinv

    Wz = jnp.dot(Wzc, Wzl[:HID, :])
    bz = jnp.dot(bzc, Wzl[:HID, :]) + bzl
    Wh = jnp.dot(Whc, Whl[:HID, :])
    bh = jnp.dot(bhc, Whl[:HID, :]) + bhl

    hsum = jnp.zeros((N, HID), jnp.float32)
    h0 = None
    hlast = None
    for t in range(SEQ):
        St = S[:, t * FIN:(t + 1) * FIN]
        Zt = jax.nn.sigmoid(jnp.dot(St, Wz) + bz)
        Tt = jnp.tanh(jnp.dot(St, Wh) + bh)
        ht = (1.0 - Zt) * Tt
        if t == 0:
            h0 = ht
        if t == SEQ - 1:
            hlast = ht
        hsum = hsum + ht

    Psum = jnp.dot(hsum, projW) + SEQ * projb
    P0 = jnp.dot(h0, projW) + projb
    PL = jnp.dot(hlast, projW) + projb
    ht_mean = (jnp.dot(Psum - PL, convk[0]) + jnp.dot(Psum, convk[1])
               + jnp.dot(Psum - P0, convk[2])) * (1.0 / SEQ) + convb

    hi = hsum * (1.0 / SEQ)
    hi = _tconv(hi, C, q1W, q1b, k1W, k1b, v1W, v1b, s1W, s1b)
    hi = jnp.maximum(_layer_norm(hi, ln1g, ln1b), 0.0)
    hi = _tconv(hi, C, q2W, q2b, k2W, k2b, v2W, v2b, s2W, s2b)
    hi = jnp.maximum(_layer_norm(hi, ln2g, ln2b), 0.0)
    hi = hi + jnp.dot(hi, skW) + skb

    hc = jnp.concatenate([ht_mean, hi], axis=1)
    npred_out[...] = jnp.dot(hc, predW) + predb
    for g in range(N // TI):
        ait_out[g] = (lax.dot_general(
            ip1W[:2 * HID, :], hc[g * TI:(g + 1) * TI, :],
            (((0,), (1,)), ((), ()))) + ip1b).astype(jnp.bfloat16)
    bjt_out[...] = lax.dot_general(
        ip1W[2 * HID:, :], hc, (((0,), (1,)), ((), ()))).astype(jnp.bfloat16)


TI = 32


GJ = 8


def _pair_body(ait, bjt, w2, b2t, w3t, b3, out):
    at = ait[0]
    bt = bjt[...]
    w2v = w2[...]
    b2v = b2t[...]
    w3v = w3t[...]
    zero = jnp.bfloat16(0.0)
    rows = []
    for g in range(TI // GJ):
        planes = [jnp.maximum(at[:, i:i + 1] + bt, zero)
                  for i in range(g * GJ, (g + 1) * GJ)]
        h1 = jnp.concatenate(planes, axis=1)
        z = lax.dot_general(w2v, h1, (((0,), (0,)), ((), ())),
                            preferred_element_type=jnp.float32)
        h2 = jnp.maximum(z + b2v, 0.0)
        rsum = jnp.sum(h2 * w3v, axis=0, keepdims=True)
        rows.extend(rsum[:, i * N:(i + 1) * N] for i in range(GJ))
    r = jnp.concatenate(rows, axis=0) + b3[0, 0]
    out[...] = jax.nn.sigmoid(r)


def kernel(x, edge_index, edge_weight, params):
    p = params
    t = p['tgcn']
    tc1 = p['tc1']
    tc2 = p['tc2']

    src = edge_index[0].reshape(E // 128, 128)
    dst = edge_index[1].reshape(E // 128, 128)
    ew = edge_weight.reshape(E // 128, 128)
    wp, cp = _edge_scatter()(src, dst, ew)
    wp = wp.reshape(N, N)
    cp = cp.reshape(N, N)

    x2d = jnp.transpose(x, (1, 0, 2)).reshape(N, SEQ * FIN)
    convk = jnp.transpose(p['conv_W'], (2, 1, 0))
    r2 = lambda v: v.reshape(1, -1)

    npred, ait, bjt = pl.pallas_call(
        _dense_body,
        out_shape=[
            jax.ShapeDtypeStruct((N, OUT), jnp.float32),
            jax.ShapeDtypeStruct((N // TI, HID, TI), jnp.bfloat16),
            jax.ShapeDtypeStruct((HID, N), jnp.bfloat16),
        ],
    )(
        wp, cp, x2d,
        t['Wzc'], t['Wzl'], r2(t['bzc']), r2(t['bzl']),
        t['Whc'], t['Whl'], r2(t['bhc']), r2(t['bhl']),
        p['proj_W'], r2(p['proj_b']), convk, r2(p['conv_b']),
        tc1['Wq'], r2(tc1['bq']), tc1['Wk'], r2(tc1['bk']),
        tc1['Wv'], r2(tc1['bv']), tc1['Ws'], r2(tc1['bs']),
        r2(p['ln1_g']), r2(p['ln1_b']),
        tc2['Wq'], r2(tc2['bq']), tc2['Wk'], r2(tc2['bk']),
        tc2['Wv'], r2(tc2['bv']), tc2['Ws'], r2(tc2['bs']),
        r2(p['ln2_g']), r2(p['ln2_b']),
        p['skip_W'], r2(p['skip_b']), p['pred_W'], r2(p['pred_b']),
        p['ip1_W'], p['ip1_b'].reshape(-1, 1),
    )

    inter = pl.pallas_call(
        _pair_body,
        grid=(N // TI,),
        in_specs=[
            pl.BlockSpec((1, HID, TI), lambda i: (i, 0, 0)),
            pl.BlockSpec((HID, N), lambda i: (0, 0)),
            pl.BlockSpec((HID, HID // 2), lambda i: (0, 0)),
            pl.BlockSpec((HID // 2, 1), lambda i: (0, 0)),
            pl.BlockSpec((HID // 2, 1), lambda i: (0, 0)),
            pl.BlockSpec((1, 1), lambda i: (0, 0)),
        ],
        out_specs=pl.BlockSpec((TI, N), lambda i: (i, 0)),
        out_shape=jax.ShapeDtypeStruct((N, N), jnp.float32),
    )(ait, bjt,
      p['ip2_W'].astype(jnp.bfloat16), p['ip2_b'].reshape(-1, 1),
      p['ip3_W'], p['ip3_b'].reshape(1, 1))

    return npred, inter
